# baseline (speedup 1.0000x reference)
import jax
import jax.numpy as jnp
from jax.experimental import pallas as pl

D = 768
RANK = 192
M_BLK = 1024


def _fused_lowrank_kernel(x_ref, u_ref, v_ref, out_ref):
    h = jnp.dot(x_ref[...], u_ref[...], preferred_element_type=jnp.float32)
    out_ref[...] = jnp.dot(h, v_ref[...], preferred_element_type=jnp.float32)


def kernel(x, U, V):
    b, s, d = x.shape
    m = b * s
    x2 = x.reshape(m, d)
    grid = (m // M_BLK,)
    out = pl.pallas_call(
        _fused_lowrank_kernel,
        grid=grid,
        in_specs=[
            pl.BlockSpec((M_BLK, d), lambda i: (i, 0)),
            pl.BlockSpec((d, RANK), lambda i: (0, 0)),
            pl.BlockSpec((RANK, d), lambda i: (0, 0)),
        ],
        out_specs=pl.BlockSpec((M_BLK, d), lambda i: (i, 0)),
        out_shape=jax.ShapeDtypeStruct((m, d), x.dtype),
    )(x2, U, V)
    return out.reshape(b, s, d)
